# Initial kernel scaffold; baseline (speedup 1.0000x reference)
import jax
import jax.numpy as jnp
from jax.experimental import pallas as pl


def kernel(z, embed):
    raise NotImplementedError("write your pallas kernel here")


# speedup vs baseline: 1.0820x; 1.0820x over previous
import functools

import jax
import jax.numpy as jnp
from jax import lax
from jax.experimental import pallas as pl
from jax.experimental.pallas import tpu as pltpu
from jax.experimental.pallas import tpu_sc as plsc

NE = 1024
ED = 64
ROWS = 8 * 32 * 32
BLK = 1024
NSTEPS = ROWS // BLK
TOT = ROWS * ED


def _vq_body(zf_ref, emb_ref, idx_ref, loss_ref, perp_ref, counts_ref, dsum_ref):
    step = pl.program_id(0)
    zb = zf_ref[...]
    emb = emb_ref[...]
    z2 = jnp.sum(zb * zb, axis=1, keepdims=True)
    e2 = jnp.sum(emb * emb, axis=1)
    mm = lax.dot_general(zb, emb, (((1,), (1,)), ((), ())),
                         preferred_element_type=jnp.float32)
    d = (z2 + e2[None, :]) - 2.0 * mm
    dmin = jnp.min(d, axis=1, keepdims=True)
    iota = lax.broadcasted_iota(jnp.int32, d.shape, 1)
    idx = jnp.min(jnp.where(d == dmin, iota, NE), axis=1)
    idx_ref[0, 0, :] = idx

    codes = lax.broadcasted_iota(jnp.int32, (BLK, NE), 1)
    hist = jnp.sum((idx[:, None] == codes).astype(jnp.float32), axis=0)

    @pl.when(step == 0)
    def _init():
        counts_ref[...] = jnp.zeros_like(counts_ref)
        dsum_ref[0] = 0.0

    counts_ref[...] += hist[None, :]
    dsum_ref[0] += jnp.sum(dmin)

    @pl.when(step == NSTEPS - 1)
    def _finish():
        m = dsum_ref[0] / float(TOT)
        loss_ref[0, 0] = 0.25 * m + m
        p = counts_ref[0, :] * (1.0 / float(ROWS))
        ent = jnp.sum(p * jnp.log(p + 1e-10))
        perp_ref[0, 0] = jnp.exp(-ent)


_vq_call = pl.pallas_call(
    _vq_body,
    grid=(NSTEPS,),
    in_specs=[
        pl.BlockSpec((BLK, ED), lambda i: (i, 0)),
        pl.BlockSpec((NE, ED), lambda i: (0, 0)),
    ],
    out_specs=[
        pl.BlockSpec((1, 1, BLK), lambda i: (i, 0, 0)),
        pl.BlockSpec(memory_space=pltpu.SMEM, block_shape=(1, 1), index_map=lambda i: (0, 0)),
        pl.BlockSpec(memory_space=pltpu.SMEM, block_shape=(1, 1), index_map=lambda i: (0, 0)),
    ],
    out_shape=[
        jax.ShapeDtypeStruct((NSTEPS, 1, BLK), jnp.int32),
        jax.ShapeDtypeStruct((1, 1), jnp.float32),
        jax.ShapeDtypeStruct((1, 1), jnp.float32),
    ],
    scratch_shapes=[
        pltpu.VMEM((1, NE), jnp.float32),
        pltpu.SMEM((1,), jnp.float32),
    ],
    compiler_params=pltpu.CompilerParams(
        dimension_semantics=("arbitrary",),
    ),
)


def kernel(z, embed):
    zf = jnp.transpose(z, (0, 2, 3, 1)).reshape(ROWS, ED)
    idx3, loss, perp = _vq_call(zf, embed)
    idx = idx3.reshape(ROWS)
    zq = jnp.take(embed, idx, axis=0)
    zq_st = zf + (zq - zf)
    zq_out = zq_st.reshape(8, 32, 32, ED).transpose(0, 3, 1, 2)
    indices = idx.reshape(8, 32, 32)
    return zq_out, indices, loss[0, 0], perp[0, 0]


# speedup vs baseline: 1.1746x; 1.0856x over previous
import functools

import jax
import jax.numpy as jnp
from jax import lax
from jax.experimental import pallas as pl
from jax.experimental.pallas import tpu as pltpu
from jax.experimental.pallas import tpu_sc as plsc

NE = 1024
ED = 64
ROWS = 8 * 32 * 32
BLK = 1024
NSTEPS = ROWS // BLK
TOT = ROWS * ED


def _vq_body(zf_ref, emb_ref, idx_ref, loss_ref, perp_ref, counts_ref, dsum_ref):
    step = pl.program_id(0)
    zb = zf_ref[...]
    emb = emb_ref[...]
    z2 = jnp.sum(zb * zb, axis=1, keepdims=True)
    e2 = jnp.sum(emb * emb, axis=1)
    mm = lax.dot_general(zb, emb, (((1,), (1,)), ((), ())),
                         preferred_element_type=jnp.float32)
    d = (z2 + e2[None, :]) - 2.0 * mm
    dmin = jnp.min(d, axis=1, keepdims=True)
    iota = lax.broadcasted_iota(jnp.int32, d.shape, 1)
    idx = jnp.min(jnp.where(d == dmin, iota, NE), axis=1)
    idx_ref[0, 0, :] = idx

    codes = lax.broadcasted_iota(jnp.int32, (BLK, NE), 1)
    hist = jnp.sum((idx[:, None] == codes).astype(jnp.float32), axis=0)

    @pl.when(step == 0)
    def _init():
        counts_ref[...] = jnp.zeros_like(counts_ref)
        dsum_ref[0] = 0.0

    counts_ref[...] += hist[None, :]
    dsum_ref[0] += jnp.sum(dmin)

    @pl.when(step == NSTEPS - 1)
    def _finish():
        m = dsum_ref[0] / float(TOT)
        loss_ref[0, 0] = 0.25 * m + m
        p = counts_ref[0, :] * (1.0 / float(ROWS))
        ent = jnp.sum(p * jnp.log(p + 1e-10))
        perp_ref[0, 0] = jnp.exp(-ent)


_vq_call = pl.pallas_call(
    _vq_body,
    grid=(NSTEPS,),
    in_specs=[
        pl.BlockSpec((BLK, ED), lambda i: (i, 0)),
        pl.BlockSpec((NE, ED), lambda i: (0, 0)),
    ],
    out_specs=[
        pl.BlockSpec((1, 1, BLK), lambda i: (i, 0, 0)),
        pl.BlockSpec(memory_space=pltpu.SMEM, block_shape=(1, 1), index_map=lambda i: (0, 0)),
        pl.BlockSpec(memory_space=pltpu.SMEM, block_shape=(1, 1), index_map=lambda i: (0, 0)),
    ],
    out_shape=[
        jax.ShapeDtypeStruct((NSTEPS, 1, BLK), jnp.int32),
        jax.ShapeDtypeStruct((1, 1), jnp.float32),
        jax.ShapeDtypeStruct((1, 1), jnp.float32),
    ],
    scratch_shapes=[
        pltpu.VMEM((1, NE), jnp.float32),
        pltpu.SMEM((1,), jnp.float32),
    ],
    compiler_params=pltpu.CompilerParams(
        dimension_semantics=("arbitrary",),
    ),
)


_NW = 32
_BPW = ROWS // _NW
_CHUNK = 128
_NCH = _BPW // _CHUNK


def _gather_body(emb_hbm, idx_hbm, out_hbm, idx_v, rows_v, sem):
    wid = lax.axis_index("s") * 2 + lax.axis_index("c")
    base = wid * _NCH
    pltpu.sync_copy(idx_hbm.at[pl.ds(base, _NCH)], idx_v)
    for j in range(_NCH):
        pltpu.async_copy(emb_hbm.at[idx_v.at[j]],
                         rows_v.at[pl.ds(j * _CHUNK, _CHUNK)], sem)
    for j in range(_NCH):
        pltpu.make_async_copy(emb_hbm.at[idx_v.at[j]],
                              rows_v.at[pl.ds(j * _CHUNK, _CHUNK)], sem).wait()
    pltpu.sync_copy(rows_v, out_hbm.at[pl.ds(wid * _BPW, _BPW)])


_gather_call = pl.kernel(
    _gather_body,
    out_type=jax.ShapeDtypeStruct((ROWS, ED), jnp.float32),
    mesh=plsc.VectorSubcoreMesh(core_axis_name="c", subcore_axis_name="s"),
    scratch_types=[
        pltpu.VMEM((_NCH, _CHUNK), jnp.int32),
        pltpu.VMEM((_BPW, ED), jnp.float32),
        pltpu.SemaphoreType.DMA,
    ],
    compiler_params=pltpu.CompilerParams(use_tc_tiling_on_sc=False),
)


def kernel(z, embed):
    zf = jnp.transpose(z, (0, 2, 3, 1)).reshape(ROWS, ED)
    idx3, loss, perp = _vq_call(zf, embed)
    idx = idx3.reshape(ROWS)
    zq = _gather_call(embed, idx3.reshape(ROWS // _CHUNK, _CHUNK))
    zq_st = zf + (zq - zf)
    zq_out = zq_st.reshape(8, 32, 32, ED).transpose(0, 3, 1, 2)
    indices = idx.reshape(8, 32, 32)
    return zq_out, indices, loss[0, 0], perp[0, 0]


# speedup vs baseline: 1.5205x; 1.2944x over previous
import jax
import jax.numpy as jnp
from jax import lax
from jax.experimental import pallas as pl
from jax.experimental.pallas import tpu as pltpu
from jax.experimental.pallas import tpu_sc as plsc

NE = 1024
ED = 64
ROWS = 8 * 32 * 32
BLK = 1024
NSTEPS = ROWS // BLK
TOT = ROWS * ED


def _vq_body(zf_ref, emb_ref, idx_ref, loss_ref, perp_ref,
             counts_ref, dsum_ref):
    step = pl.program_id(0)
    zb = zf_ref[...]
    emb = emb_ref[...]
    z2 = jnp.sum(zb * zb, axis=1, keepdims=True)
    e2 = jnp.sum(emb * emb, axis=1)
    mmn = lax.dot_general(-2.0 * zb, emb, (((1,), (1,)), ((), ())),
                          preferred_element_type=jnp.float32)
    d = (z2 + e2[None, :]) + mmn
    dmin = jnp.min(d, axis=1, keepdims=True)
    iota_f = lax.broadcasted_iota(jnp.int32, d.shape, 1).astype(jnp.float32)
    onehot = jnp.where(d == dmin, iota_f, float(2 * NE))
    idx_f = jnp.min(onehot, axis=1, keepdims=True)
    idx = idx_f[:, 0].astype(jnp.int32)
    idx_ref[0, 0, :] = idx

    oneh = (idx_f == iota_f).astype(jnp.float32)
    hist = lax.dot_general(jnp.ones((1, BLK), jnp.float32), oneh,
                           (((1,), (0,)), ((), ())),
                           preferred_element_type=jnp.float32)[0]

    @pl.when(step == 0)
    def _init():
        counts_ref[...] = jnp.zeros_like(counts_ref)
        dsum_ref[0] = 0.0

    counts_ref[...] += hist[None, :]
    dsum_ref[0] += jnp.sum(dmin)

    @pl.when(step == NSTEPS - 1)
    def _finish():
        m = dsum_ref[0] / float(TOT)
        loss_ref[0, 0] = 0.25 * m + m
        p = counts_ref[0, :] * (1.0 / float(ROWS))
        ent = jnp.sum(p * jnp.log(p + 1e-10))
        perp_ref[0, 0] = jnp.exp(-ent)


_vq_call = pl.pallas_call(
    _vq_body,
    grid=(NSTEPS,),
    in_specs=[
        pl.BlockSpec((BLK, ED), lambda i: (i, 0)),
        pl.BlockSpec((NE, ED), lambda i: (0, 0)),
    ],
    out_specs=[
        pl.BlockSpec((1, 1, BLK), lambda i: (i, 0, 0)),
        pl.BlockSpec(memory_space=pltpu.SMEM, block_shape=(1, 1), index_map=lambda i: (0, 0)),
        pl.BlockSpec(memory_space=pltpu.SMEM, block_shape=(1, 1), index_map=lambda i: (0, 0)),
    ],
    out_shape=[
        jax.ShapeDtypeStruct((NSTEPS, 1, BLK), jnp.int32),
        jax.ShapeDtypeStruct((1, 1), jnp.float32),
        jax.ShapeDtypeStruct((1, 1), jnp.float32),
    ],
    scratch_shapes=[
        pltpu.VMEM((1, NE), jnp.float32),
        pltpu.SMEM((1,), jnp.float32),
    ],
    compiler_params=pltpu.CompilerParams(
        dimension_semantics=("arbitrary",),
    ),
)


_NC = 1
_NW = _NC * 16
_BPW = ROWS // _NW
_CHUNK = 128
_NCH = _BPW // _CHUNK


def _gather_body(emb_hbm, idx_hbm, out_hbm, idx_v, rows_v, sem):
    wid = lax.axis_index("s") * _NC + lax.axis_index("c")
    base = wid * _NCH
    pltpu.sync_copy(idx_hbm.at[pl.ds(base, _NCH)], idx_v)
    for j in range(_NCH):
        pltpu.async_copy(emb_hbm.at[idx_v.at[j]],
                         rows_v.at[pl.ds(j * _CHUNK, _CHUNK)], sem)
    for j in range(_NCH):
        pltpu.make_async_copy(emb_hbm.at[idx_v.at[j]],
                              rows_v.at[pl.ds(j * _CHUNK, _CHUNK)], sem).wait()
    pltpu.sync_copy(rows_v, out_hbm.at[pl.ds(wid * _BPW, _BPW)])


def _gather(embed, idx2d):
    call = pl.kernel(
        _gather_body,
        out_type=jax.ShapeDtypeStruct((ROWS, ED), jnp.float32),
        mesh=plsc.VectorSubcoreMesh(core_axis_name="c", subcore_axis_name="s",
                                    num_cores=_NC),
        scratch_types=[
            pltpu.VMEM((_NCH, _CHUNK), jnp.int32),
            pltpu.VMEM((_BPW, ED), jnp.float32),
            pltpu.SemaphoreType.DMA,
        ],
        compiler_params=pltpu.CompilerParams(use_tc_tiling_on_sc=False),
    )
    return call(embed, idx2d)


def kernel(z, embed):
    zf = jnp.transpose(z, (0, 2, 3, 1)).reshape(ROWS, ED)
    idx3, loss, perp = _vq_call(zf, embed)
    zq = _gather(embed, idx3.reshape(ROWS // _CHUNK, _CHUNK))
    zq_out = zq.reshape(8, 32, 32, ED).transpose(0, 3, 1, 2)
    indices = idx3.reshape(8, 32, 32)
    return zq_out, indices, loss[0, 0], perp[0, 0]
